# baseline (speedup 1.0000x reference)
import jax
import jax.numpy as jnp
from jax import lax
from jax.experimental import pallas as pl
from jax.experimental.pallas import tpu as pltpu
from jax.experimental.pallas import tpu_sc as plsc

_N = 50000
_E = 800000
_RNA = 256
_PROT = 256
_EMB = 64
_HID = 16
_H = 4
_C = 16

_BN = 1000
_EPT = _E // 16
_W = 400
_WINS = _EPT // _W
assert _EPT % _W == 0 and _W % 16 == 0

_RQ = 3120
_REXT = _N - 16 * _RQ
_ZR = 40
_ZRF = 520


def _tc1_body(x_ref, We, be, Wb, bl, ab_ref, xsp_ref, hl_ref, mx_ref):
    i = pl.program_id(0)
    h = x_ref[...] @ We[...] + be[...]
    q = h @ Wb[...]
    xs = q[:, :64]
    hl_ref[...] = q[:, 64:128] + bl[...]
    ab = q[:, 128:]
    ab_ref[...] = ab.reshape(_BN * 128)
    xsp_ref[...] = jnp.concatenate([xs, jnp.zeros((_BN, 64), jnp.float32)], 1)
    m = ab[:, :8].max(0).reshape(1, 8)

    @pl.when(i == 0)
    def _init():
        mx_ref[...] = m

    @pl.when(i > 0)
    def _acc():
        mx_ref[...] = jnp.maximum(mx_ref[...], m)


def _tc1(x, We, be, Wb, bl):
    grid = _N // _BN
    full = lambda shape: pl.BlockSpec(shape, lambda i: tuple(0 for _ in shape))
    return pl.pallas_call(
        _tc1_body,
        grid=(grid,),
        in_specs=[
            pl.BlockSpec((_BN, 2 * _RNA), lambda i: (i, 0)),
            full((2 * _RNA, 2 * _EMB)), full((1, 2 * _EMB)),
            full((2 * _EMB, 256)), full((1, 64)),
        ],
        out_specs=[
            pl.BlockSpec((128 * _BN,), lambda i: (i,)),
            pl.BlockSpec((_BN, 128), lambda i: (i, 0)),
            pl.BlockSpec((_BN, 64), lambda i: (i, 0)),
            pl.BlockSpec((1, 8), lambda i: (0, 0)),
        ],
        out_shape=[
            jax.ShapeDtypeStruct((128 * _N,), jnp.float32),
            jax.ShapeDtypeStruct((_N, 128), jnp.float32),
            jax.ShapeDtypeStruct((_N, 64), jnp.float32),
            jax.ShapeDtypeStruct((1, 8), jnp.float32),
        ],
    )(x, We, be, Wb, bl)


def _tc2_body(gat_ref, hl_ref, bg, Wb, bl, ab_ref, xsp_ref, hl2_ref, mx_ref):
    i = pl.program_id(0)
    g64 = jnp.concatenate([gat_ref[0], gat_ref[1]], axis=1)
    h = jax.nn.relu(g64 + bg[...] + hl_ref[...])
    q = h @ Wb[...]
    xs = q[:, :64]
    hl2_ref[...] = q[:, 64:128] + bl[...]
    ab = q[:, 128:]
    ab_ref[...] = ab.reshape(_BN * 128)
    xsp_ref[...] = jnp.concatenate([xs, jnp.zeros((_BN, 64), jnp.float32)], 1)
    m = ab[:, :8].max(0).reshape(1, 8)

    @pl.when(i == 0)
    def _init():
        mx_ref[...] = m

    @pl.when(i > 0)
    def _acc():
        mx_ref[...] = jnp.maximum(mx_ref[...], m)


def _tc2(gat, hl, bg, Wb, bl):
    grid = _N // _BN
    full = lambda shape: pl.BlockSpec(shape, lambda i: tuple(0 for _ in shape))
    return pl.pallas_call(
        _tc2_body,
        grid=(grid,),
        in_specs=[
            pl.BlockSpec((2, _BN, 32), lambda i: (0, i, 0)),
            pl.BlockSpec((_BN, 64), lambda i: (i, 0)),
            full((1, 64)),
            full((_H * _C, 256)), full((1, 64)),
        ],
        out_specs=[
            pl.BlockSpec((128 * _BN,), lambda i: (i,)),
            pl.BlockSpec((_BN, 128), lambda i: (i, 0)),
            pl.BlockSpec((_BN, 64), lambda i: (i, 0)),
            pl.BlockSpec((1, 8), lambda i: (0, 0)),
        ],
        out_shape=[
            jax.ShapeDtypeStruct((128 * _N,), jnp.float32),
            jax.ShapeDtypeStruct((_N, 128), jnp.float32),
            jax.ShapeDtypeStruct((_N, 64), jnp.float32),
            jax.ShapeDtypeStruct((1, 8), jnp.float32),
        ],
    )(gat, hl, bg, Wb, bl)


def _tc3_body(gat_ref, hl_ref, bg, Wagg, bagg, Wd, bd, Wr, br,
              rr_ref, pr_ref, emb_ref):
    g64 = jnp.concatenate([gat_ref[0], gat_ref[1]], axis=1)
    h = jax.nn.relu(g64 + bg[...] + hl_ref[...])
    emb = jax.nn.relu(h @ Wagg[...] + bagg[...])
    dcat = emb @ Wd[...] + bd[...]
    recon = dcat @ Wr[...] + br[...]
    rr_ref[...] = recon[:, :_RNA]
    pr_ref[...] = recon[:, _RNA:]
    emb_ref[...] = emb


def _tc3(gat, hl, bg, Wagg, bagg, Wd, bd, Wr, br):
    grid = _N // _BN
    full = lambda shape: pl.BlockSpec(shape, lambda i: tuple(0 for _ in shape))
    return pl.pallas_call(
        _tc3_body,
        grid=(grid,),
        in_specs=[
            pl.BlockSpec((2, _BN, 32), lambda i: (0, i, 0)),
            pl.BlockSpec((_BN, 64), lambda i: (i, 0)),
            full((1, 64)),
            full((_H * _C, _HID)), full((1, _HID)),
            full((_HID, 2 * _EMB)), full((1, 2 * _EMB)),
            full((2 * _EMB, _RNA + _PROT)), full((1, _RNA + _PROT)),
        ],
        out_specs=[
            pl.BlockSpec((_BN, _RNA), lambda i: (i, 0)),
            pl.BlockSpec((_BN, _PROT), lambda i: (i, 0)),
            pl.BlockSpec((_BN, _HID), lambda i: (i, 0)),
        ],
        out_shape=[
            jax.ShapeDtypeStruct((_N, _RNA), jnp.float32),
            jax.ShapeDtypeStruct((_N, _PROT), jnp.float32),
            jax.ShapeDtypeStruct((_N, _HID), jnp.float32),
        ],
    )(gat, hl, bg, Wagg, bagg, Wd, bd, Wr, br)


def _vperm(vec, idx16):
    return lax.gather(
        vec, idx16[:, None],
        lax.GatherDimensionNumbers(offset_dims=(), collapsed_slice_dims=(0,),
                                   start_index_map=(0,)),
        (1,), mode=lax.GatherScatterMode.PROMISE_IN_BOUNDS)


def _sc_body(src_h, dst_h, ab_h, xs4_h, mt_h, zn_h, zd_h,
             gat_h,
             src_v, dst_v, sadj_v, elsall_v, deni_v,
             albf, exf, xsg, mt_v,
             num_sh, den_sh, sem0, sem2, sem3, sem4):
    c = lax.axis_index("c")
    s = lax.axis_index("s")
    lanes = lax.iota(jnp.int32, 16)
    half = lanes >> 1
    pof = lanes & 1

    pltpu.sync_copy(zn_h.at[pl.ds(0, _RQ)], num_sh.at[pl.ds(s * _RQ, _RQ)])
    pltpu.sync_copy(zd_h.at[pl.ds(0, 2 * _RQ)],
                    den_sh.at[pl.ds(2 * s * _RQ, 2 * _RQ)])

    @pl.when(s == 15)
    def _ztail():
        pltpu.sync_copy(zn_h.at[pl.ds(0, _REXT)],
                        num_sh.at[pl.ds(16 * _RQ, _REXT)])
        pltpu.sync_copy(zd_h.at[pl.ds(0, 2 * _REXT)],
                        den_sh.at[pl.ds(32 * _RQ, 2 * _REXT)])

    pltpu.sync_copy(mt_h.at[c], mt_v)
    plsc.subcore_barrier()
    mtv = mt_v[...]

    def _prefetch(w):
        base = (s * _WINS + w) * _W
        par = w & 1
        pltpu.sync_copy(src_h.at[pl.ds(base, _W)], src_v)
        pltpu.sync_copy(dst_h.at[pl.ds(base, _W)], dst_v.at[par])

        def _idx(j):
            s16 = src_v[pl.ds(j * 16, 16)]
            d16 = dst_v[par, pl.ds(j * 16, 16)]
            sadj_v[par, pl.ds(j * 16, 16)] = s16 * 4 + c
            av = s16 * 128 + 2 * c
            bv = d16 * 128 + (4 + 2 * c)
            dv = d16 * 2
            for q in range(2):
                pq = half + q * 8
                off = pl.ds(j * 32 + q * 16, 16)
                off2 = pl.ds(2 * _W + j * 32 + q * 16, 16)
                elsall_v[par, off] = _vperm(av, pq) + pof
                elsall_v[par, off2] = _vperm(bv, pq) + pof
                deni_v[par, off] = _vperm(dv, pq) + pof
        plsc.parallel_loop(0, _W // 16, 1, unroll=2)(_idx)
        pltpu.async_copy(ab_h.at[elsall_v.at[par]], albf.at[par], sem0)

    _prefetch(0)

    def _window(w, _):
        par = w & 1

        @pl.when(w > 0)
        def _drain_num():
            pltpu.make_async_copy(xsg, num_sh.at[dst_v.at[1 - par]],
                                  sem4).wait()
        cp_x = pltpu.async_copy(xs4_h.at[sadj_v.at[par]], xsg, sem2)

        pltpu.make_async_copy(ab_h.at[elsall_v.at[par]], albf.at[par],
                              sem0).wait()

        @pl.when(w > 0)
        def _drain_den():
            pltpu.make_async_copy(exf, den_sh.at[deni_v.at[1 - par]],
                                  sem3).wait()

        def _ex(j):
            a = albf[par, pl.ds(j * 16, 16)]
            b = albf[par, pl.ds(2 * _W + j * 16, 16)]
            t = a + b
            e = jnp.maximum(t, 0.2 * t)
            exf[pl.ds(j * 16, 16)] = jnp.exp(e - mtv)
        plsc.parallel_loop(0, 2 * _W // 16, 1, unroll=4)(_ex)

        pltpu.async_copy(exf, den_sh.at[deni_v.at[par]], sem3, add=True)

        @pl.when(w + 1 < _WINS)
        def _pref():
            _prefetch(w + 1)

        cp_x.wait()

        def _scale(g):
            ex16 = exf[pl.ds(g * 16, 16)]
            for k in range(16):
                w_e = g * 8 + (k >> 1)
                hp = k & 1
                mult = _vperm(ex16, lanes * 0 + k)
                xv = xsg[w_e, pl.ds(hp * 16, 16)]
                xsg[w_e, pl.ds(hp * 16, 16)] = xv * mult
        plsc.parallel_loop(0, _W // 8, 1, unroll=2)(_scale)

        pltpu.async_copy(xsg, num_sh.at[dst_v.at[par]], sem4, add=True)
        return 0

    lax.fori_loop(0, _WINS, _window, 0)

    lpar = (_WINS - 1) & 1
    pltpu.make_async_copy(exf, den_sh.at[deni_v.at[lpar]], sem3).wait()
    pltpu.make_async_copy(xsg, num_sh.at[dst_v.at[lpar]], sem4).wait()

    plsc.subcore_barrier()

    def _norm_chunk(rowbase, nrows):
        pltpu.sync_copy(num_sh.at[pl.ds(rowbase, nrows)],
                        xsg.at[pl.ds(0, nrows)])
        pltpu.sync_copy(den_sh.at[pl.ds(2 * rowbase, 2 * nrows)],
                        exf.at[pl.ds(0, 2 * nrows)])

        def _nrm(j, _):
            d16 = exf[pl.ds(j * 16, 16)] + 1e-16
            for k in range(8):
                row = j * 8 + k
                dv0 = _vperm(d16, lanes * 0 + 2 * k)
                dv1 = _vperm(d16, lanes * 0 + 2 * k + 1)
                xsg[row, pl.ds(0, 16)] = xsg[row, pl.ds(0, 16)] / dv0
                xsg[row, pl.ds(16, 16)] = xsg[row, pl.ds(16, 16)] / dv1
            return 0
        lax.fori_loop(0, 2 * nrows // 16, _nrm, 0)
        pltpu.sync_copy(xsg.at[pl.ds(0, nrows)],
                        gat_h.at[c, pl.ds(rowbase, nrows)])

    def _wchunk(k, _):
        _norm_chunk(s * _RQ + k * 240, 240)
        return 0
    lax.fori_loop(0, _RQ // 240, _wchunk, 0)

    @pl.when(s == 15)
    def _wtail():
        _norm_chunk(16 * _RQ, _REXT)


def _sc_layer(src, dst, ab, xs4, mt, zn, zd):
    mesh = plsc.VectorSubcoreMesh(core_axis_name="c", subcore_axis_name="s")
    f = pl.kernel(
        _sc_body,
        mesh=mesh,
        out_type=jax.ShapeDtypeStruct((2, _N, 32), jnp.float32),
        scratch_types=[
            pltpu.VMEM((_W,), jnp.int32),
            pltpu.VMEM((2, _W), jnp.int32),
            pltpu.VMEM((2, _W), jnp.int32),
            pltpu.VMEM((2, 4 * _W), jnp.int32),
            pltpu.VMEM((2, 2 * _W), jnp.int32),
            pltpu.VMEM((2, 4 * _W), jnp.float32),
            pltpu.VMEM((2 * _W,), jnp.float32),
            pltpu.VMEM((_W, 32), jnp.float32),
            pltpu.VMEM((16,), jnp.float32),
            pltpu.VMEM_SHARED((_N, 32), jnp.float32),
            pltpu.VMEM_SHARED((2 * _N,), jnp.float32),
            pltpu.SemaphoreType.DMA,
            pltpu.SemaphoreType.DMA,
            pltpu.SemaphoreType.DMA,
            pltpu.SemaphoreType.DMA,
        ],
        compiler_params=pltpu.CompilerParams(use_tc_tiling_on_sc=False),
    )
    return f(src, dst, ab, xs4, mt, zn, zd)


def _mt_from_mx(mx):
    t = mx[0, :4] + mx[0, 4:]
    m = jnp.maximum(t, 0.2 * t)
    return jnp.tile(m.reshape(2, 2), (1, 8))


def _proj_weights(Ws, Wl, a_s, a_d, Wd):
    k = Ws.shape[0]
    A_s = (Ws.reshape(k, _H, _C) * a_s[None]).sum(-1)
    A_d = (Wd.reshape(k, _H, _C) * a_d[None]).sum(-1)
    ab = jnp.tile(jnp.concatenate([A_s, A_d], axis=1), (1, 16))
    return jnp.concatenate([Ws, Wl, ab], axis=1)


def kernel(x, edge_index, W_rna, b_rna, W_prot, b_prot, Ws1, Wd1, as1, ad1,
           bg1, Wl1, bl1, Ws2, Wd2, as2, ad2, bg2, Wl2, bl2, Wagg, bagg,
           Wrd, brd, Wpd, bpd, Wrr, brr, Wpr, bpr):
    src = edge_index[0]
    dst = edge_index[1]
    r2 = lambda v: v.reshape(1, -1)
    f32 = jnp.float32

    z = jnp.zeros((_RNA, _EMB), f32)
    We = jnp.concatenate([
        jnp.concatenate([W_rna, z], axis=1),
        jnp.concatenate([z, W_prot], axis=1)], axis=0)
    be = r2(jnp.concatenate([b_rna, b_prot]))
    Wb1 = _proj_weights(Ws1, Wl1, as1, ad1, Wd1)
    Wb2 = _proj_weights(Ws2, Wl2, as2, ad2, Wd2)
    Wdcat = jnp.concatenate([Wrd, Wpd], axis=1)
    bdcat = r2(jnp.concatenate([brd, bpd]))
    zr = jnp.zeros((_EMB, _RNA), f32)
    zp = jnp.zeros((_EMB, _PROT), f32)
    Wrec = jnp.concatenate([
        jnp.concatenate([Wrr, zp], axis=1),
        jnp.concatenate([zr, Wpr], axis=1)], axis=0)
    brec = r2(jnp.concatenate([brr, bpr]))

    zn = jnp.zeros((3200, 32), f32)
    zd = jnp.zeros((6400,), f32)
    ab1, xsp1, hl1, mx1 = _tc1(x, We, be, Wb1, r2(bl1))
    gat1 = _sc_layer(src, dst, ab1, xsp1.reshape(4 * _N, 32),
                     _mt_from_mx(mx1), zn, zd)

    ab2, xsp2, hl2, mx2 = _tc2(gat1, hl1, r2(bg1), Wb2, r2(bl2))
    gat2 = _sc_layer(src, dst, ab2, xsp2.reshape(4 * _N, 32),
                     _mt_from_mx(mx2), zn, zd)

    rr, pr, emb = _tc3(gat2, hl2, r2(bg2), Wagg, r2(bagg), Wdcat, bdcat,
                       Wrec, brec)
    return (rr, pr, emb)


# speedup vs baseline: 1.1703x; 1.1703x over previous
import jax
import jax.numpy as jnp
from jax import lax
from jax.experimental import pallas as pl
from jax.experimental.pallas import tpu as pltpu
from jax.experimental.pallas import tpu_sc as plsc

_N = 50000
_E = 800000
_RNA = 256
_PROT = 256
_EMB = 64
_HID = 16
_H = 4
_C = 16

_BN = 1000
_EPT = _E // 16
_W = 400
_WINS = _EPT // _W
assert _EPT % _W == 0 and _W % 16 == 0

_RQ = 3120
_REXT = _N - 16 * _RQ
_ZR = 40
_ZRF = 520


def _tc1_body(x_ref, We, be, Wb, bl, ab_ref, xsp_ref, hl_ref, mx_ref):
    i = pl.program_id(0)
    h = x_ref[...] @ We[...] + be[...]
    q = h @ Wb[...]
    xs = q[:, :64]
    hl_ref[...] = q[:, 64:128] + bl[...]
    ab = q[:, 128:]
    ab_ref[...] = ab.reshape(_BN * 128)
    xsp_ref[...] = jnp.concatenate([xs, jnp.zeros((_BN, 64), jnp.float32)], 1)
    m = ab[:, :8].max(0).reshape(1, 8)

    @pl.when(i == 0)
    def _init():
        mx_ref[...] = m

    @pl.when(i > 0)
    def _acc():
        mx_ref[...] = jnp.maximum(mx_ref[...], m)


def _tc1(x, We, be, Wb, bl):
    grid = _N // _BN
    full = lambda shape: pl.BlockSpec(shape, lambda i: tuple(0 for _ in shape))
    return pl.pallas_call(
        _tc1_body,
        grid=(grid,),
        in_specs=[
            pl.BlockSpec((_BN, 2 * _RNA), lambda i: (i, 0)),
            full((2 * _RNA, 2 * _EMB)), full((1, 2 * _EMB)),
            full((2 * _EMB, 256)), full((1, 64)),
        ],
        out_specs=[
            pl.BlockSpec((128 * _BN,), lambda i: (i,)),
            pl.BlockSpec((_BN, 128), lambda i: (i, 0)),
            pl.BlockSpec((_BN, 64), lambda i: (i, 0)),
            pl.BlockSpec((1, 8), lambda i: (0, 0)),
        ],
        out_shape=[
            jax.ShapeDtypeStruct((128 * _N,), jnp.float32),
            jax.ShapeDtypeStruct((_N, 128), jnp.float32),
            jax.ShapeDtypeStruct((_N, 64), jnp.float32),
            jax.ShapeDtypeStruct((1, 8), jnp.float32),
        ],
    )(x, We, be, Wb, bl)


def _tc2_body(gat_ref, hl_ref, bg, Wb, bl, ab_ref, xsp_ref, hl2_ref, mx_ref):
    i = pl.program_id(0)
    g64 = jnp.concatenate([gat_ref[0], gat_ref[1]], axis=1)
    h = jax.nn.relu(g64 + bg[...] + hl_ref[...])
    q = h @ Wb[...]
    xs = q[:, :64]
    hl2_ref[...] = q[:, 64:128] + bl[...]
    ab = q[:, 128:]
    ab_ref[...] = ab.reshape(_BN * 128)
    xsp_ref[...] = jnp.concatenate([xs, jnp.zeros((_BN, 64), jnp.float32)], 1)
    m = ab[:, :8].max(0).reshape(1, 8)

    @pl.when(i == 0)
    def _init():
        mx_ref[...] = m

    @pl.when(i > 0)
    def _acc():
        mx_ref[...] = jnp.maximum(mx_ref[...], m)


def _tc2(gat, hl, bg, Wb, bl):
    grid = _N // _BN
    full = lambda shape: pl.BlockSpec(shape, lambda i: tuple(0 for _ in shape))
    return pl.pallas_call(
        _tc2_body,
        grid=(grid,),
        in_specs=[
            pl.BlockSpec((2, _BN, 32), lambda i: (0, i, 0)),
            pl.BlockSpec((_BN, 64), lambda i: (i, 0)),
            full((1, 64)),
            full((_H * _C, 256)), full((1, 64)),
        ],
        out_specs=[
            pl.BlockSpec((128 * _BN,), lambda i: (i,)),
            pl.BlockSpec((_BN, 128), lambda i: (i, 0)),
            pl.BlockSpec((_BN, 64), lambda i: (i, 0)),
            pl.BlockSpec((1, 8), lambda i: (0, 0)),
        ],
        out_shape=[
            jax.ShapeDtypeStruct((128 * _N,), jnp.float32),
            jax.ShapeDtypeStruct((_N, 128), jnp.float32),
            jax.ShapeDtypeStruct((_N, 64), jnp.float32),
            jax.ShapeDtypeStruct((1, 8), jnp.float32),
        ],
    )(gat, hl, bg, Wb, bl)


def _tc3_body(gat_ref, hl_ref, bg, Wagg, bagg, Wd, bd, Wr, br,
              rr_ref, pr_ref, emb_ref):
    g64 = jnp.concatenate([gat_ref[0], gat_ref[1]], axis=1)
    h = jax.nn.relu(g64 + bg[...] + hl_ref[...])
    emb = jax.nn.relu(h @ Wagg[...] + bagg[...])
    dcat = emb @ Wd[...] + bd[...]
    recon = dcat @ Wr[...] + br[...]
    rr_ref[...] = recon[:, :_RNA]
    pr_ref[...] = recon[:, _RNA:]
    emb_ref[...] = emb


def _tc3(gat, hl, bg, Wagg, bagg, Wd, bd, Wr, br):
    grid = _N // _BN
    full = lambda shape: pl.BlockSpec(shape, lambda i: tuple(0 for _ in shape))
    return pl.pallas_call(
        _tc3_body,
        grid=(grid,),
        in_specs=[
            pl.BlockSpec((2, _BN, 32), lambda i: (0, i, 0)),
            pl.BlockSpec((_BN, 64), lambda i: (i, 0)),
            full((1, 64)),
            full((_H * _C, _HID)), full((1, _HID)),
            full((_HID, 2 * _EMB)), full((1, 2 * _EMB)),
            full((2 * _EMB, _RNA + _PROT)), full((1, _RNA + _PROT)),
        ],
        out_specs=[
            pl.BlockSpec((_BN, _RNA), lambda i: (i, 0)),
            pl.BlockSpec((_BN, _PROT), lambda i: (i, 0)),
            pl.BlockSpec((_BN, _HID), lambda i: (i, 0)),
        ],
        out_shape=[
            jax.ShapeDtypeStruct((_N, _RNA), jnp.float32),
            jax.ShapeDtypeStruct((_N, _PROT), jnp.float32),
            jax.ShapeDtypeStruct((_N, _HID), jnp.float32),
        ],
    )(gat, hl, bg, Wagg, bagg, Wd, bd, Wr, br)


def _vperm(vec, idx16):
    return lax.gather(
        vec, idx16[:, None],
        lax.GatherDimensionNumbers(offset_dims=(), collapsed_slice_dims=(0,),
                                   start_index_map=(0,)),
        (1,), mode=lax.GatherScatterMode.PROMISE_IN_BOUNDS)


def _sc_body(src_h, dst_h, ab_h, xs4_h, mt_h, zn_h, zd_h,
             gat_h,
             src_v, dst_v, sadj_v, els_v, eldt_v, deni_v,
             alsf, aldf, exf, xsg, mt_v,
             num_sh, den_sh, sem0, sem1, sem2, sem3, sem4):
    c = lax.axis_index("c")
    s = lax.axis_index("s")
    lanes = lax.iota(jnp.int32, 16)
    half = lanes >> 1
    pof = lanes & 1

    pltpu.sync_copy(zn_h.at[pl.ds(0, _RQ)], num_sh.at[pl.ds(s * _RQ, _RQ)])
    pltpu.sync_copy(zd_h.at[pl.ds(0, 2 * _RQ)],
                    den_sh.at[pl.ds(2 * s * _RQ, 2 * _RQ)])

    @pl.when(s == 15)
    def _ztail():
        pltpu.sync_copy(zn_h.at[pl.ds(0, _REXT)],
                        num_sh.at[pl.ds(16 * _RQ, _REXT)])
        pltpu.sync_copy(zd_h.at[pl.ds(0, 2 * _REXT)],
                        den_sh.at[pl.ds(32 * _RQ, 2 * _REXT)])

    pltpu.sync_copy(mt_h.at[c], mt_v)
    plsc.subcore_barrier()
    mtv = mt_v[...]

    def _prefetch(w):
        base = (s * _WINS + w) * _W
        par = w & 1
        pltpu.sync_copy(src_h.at[pl.ds(base, _W)], src_v)
        pltpu.sync_copy(dst_h.at[pl.ds(base, _W)], dst_v.at[par])

        def _idx(j):
            s16 = src_v[pl.ds(j * 16, 16)]
            d16 = dst_v[par, pl.ds(j * 16, 16)]
            sadj_v[par, pl.ds(j * 16, 16)] = s16 * 4 + c
            av = s16 * 128 + 2 * c
            bv = d16 * 128 + (4 + 2 * c)
            dv = d16 * 2
            for q in range(2):
                pq = half + q * 8
                off = pl.ds(j * 32 + q * 16, 16)
                els_v[par, off] = _vperm(av, pq) + pof
                eldt_v[par, off] = _vperm(bv, pq) + pof
                deni_v[par, off] = _vperm(dv, pq) + pof
        plsc.parallel_loop(0, _W // 16, 1, unroll=2)(_idx)
        pltpu.async_copy(ab_h.at[els_v.at[par]], alsf.at[par], sem0)
        pltpu.async_copy(ab_h.at[eldt_v.at[par]], aldf.at[par], sem1)

    _prefetch(0)

    def _window(w, _):
        par = w & 1

        @pl.when(w > 0)
        def _drain_num():
            pltpu.make_async_copy(xsg, num_sh.at[dst_v.at[1 - par]],
                                  sem4).wait()
        cp_x = pltpu.async_copy(xs4_h.at[sadj_v.at[par]], xsg, sem2)

        pltpu.make_async_copy(ab_h.at[els_v.at[par]], alsf.at[par],
                              sem0).wait()
        pltpu.make_async_copy(ab_h.at[eldt_v.at[par]], aldf.at[par],
                              sem1).wait()

        @pl.when(w > 0)
        def _drain_den():
            pltpu.make_async_copy(exf, den_sh.at[deni_v.at[1 - par]],
                                  sem3).wait()

        def _ex(j):
            a = alsf[par, pl.ds(j * 16, 16)]
            b = aldf[par, pl.ds(j * 16, 16)]
            t = a + b
            e = jnp.maximum(t, 0.2 * t)
            exf[pl.ds(j * 16, 16)] = jnp.exp(e - mtv)
        plsc.parallel_loop(0, 2 * _W // 16, 1, unroll=4)(_ex)

        pltpu.async_copy(exf, den_sh.at[deni_v.at[par]], sem3, add=True)

        @pl.when(w + 1 < _WINS)
        def _pref():
            _prefetch(w + 1)

        cp_x.wait()

        def _scale(g):
            ex16 = exf[pl.ds(g * 16, 16)]
            for k in range(16):
                w_e = g * 8 + (k >> 1)
                hp = k & 1
                mult = _vperm(ex16, lanes * 0 + k)
                xv = xsg[w_e, pl.ds(hp * 16, 16)]
                xsg[w_e, pl.ds(hp * 16, 16)] = xv * mult
        plsc.parallel_loop(0, _W // 8, 1, unroll=2)(_scale)

        pltpu.async_copy(xsg, num_sh.at[dst_v.at[par]], sem4, add=True)
        return 0

    lax.fori_loop(0, _WINS, _window, 0)

    lpar = (_WINS - 1) & 1
    pltpu.make_async_copy(exf, den_sh.at[deni_v.at[lpar]], sem3).wait()
    pltpu.make_async_copy(xsg, num_sh.at[dst_v.at[lpar]], sem4).wait()

    plsc.subcore_barrier()

    def _norm_chunk(rowbase, nrows):
        pltpu.sync_copy(num_sh.at[pl.ds(rowbase, nrows)],
                        xsg.at[pl.ds(0, nrows)])
        pltpu.sync_copy(den_sh.at[pl.ds(2 * rowbase, 2 * nrows)],
                        exf.at[pl.ds(0, 2 * nrows)])

        def _nrm(j, _):
            d16 = exf[pl.ds(j * 16, 16)] + 1e-16
            for k in range(8):
                row = j * 8 + k
                dv0 = _vperm(d16, lanes * 0 + 2 * k)
                dv1 = _vperm(d16, lanes * 0 + 2 * k + 1)
                xsg[row, pl.ds(0, 16)] = xsg[row, pl.ds(0, 16)] / dv0
                xsg[row, pl.ds(16, 16)] = xsg[row, pl.ds(16, 16)] / dv1
            return 0
        lax.fori_loop(0, 2 * nrows // 16, _nrm, 0)
        pltpu.sync_copy(xsg.at[pl.ds(0, nrows)],
                        gat_h.at[c, pl.ds(rowbase, nrows)])

    def _wchunk(k, _):
        _norm_chunk(s * _RQ + k * 240, 240)
        return 0
    lax.fori_loop(0, _RQ // 240, _wchunk, 0)

    @pl.when(s == 15)
    def _wtail():
        _norm_chunk(16 * _RQ, _REXT)


def _sc_layer(src, dst, ab, xs4, mt, zn, zd):
    mesh = plsc.VectorSubcoreMesh(core_axis_name="c", subcore_axis_name="s")
    f = pl.kernel(
        _sc_body,
        mesh=mesh,
        out_type=jax.ShapeDtypeStruct((2, _N, 32), jnp.float32),
        scratch_types=[
            pltpu.VMEM((_W,), jnp.int32),
            pltpu.VMEM((2, _W), jnp.int32),
            pltpu.VMEM((2, _W), jnp.int32),
            pltpu.VMEM((2, 2 * _W), jnp.int32),
            pltpu.VMEM((2, 2 * _W), jnp.int32),
            pltpu.VMEM((2, 2 * _W), jnp.int32),
            pltpu.VMEM((2, 2 * _W), jnp.float32),
            pltpu.VMEM((2, 2 * _W), jnp.float32),
            pltpu.VMEM((2 * _W,), jnp.float32),
            pltpu.VMEM((_W, 32), jnp.float32),
            pltpu.VMEM((16,), jnp.float32),
            pltpu.VMEM_SHARED((_N, 32), jnp.float32),
            pltpu.VMEM_SHARED((2 * _N,), jnp.float32),
            pltpu.SemaphoreType.DMA,
            pltpu.SemaphoreType.DMA,
            pltpu.SemaphoreType.DMA,
            pltpu.SemaphoreType.DMA,
            pltpu.SemaphoreType.DMA,
        ],
        compiler_params=pltpu.CompilerParams(use_tc_tiling_on_sc=False),
    )
    return f(src, dst, ab, xs4, mt, zn, zd)


def _mt_from_mx(mx):
    t = mx[0, :4] + mx[0, 4:]
    m = jnp.maximum(t, 0.2 * t)
    return jnp.tile(m.reshape(2, 2), (1, 8))


def _proj_weights(Ws, Wl, a_s, a_d, Wd):
    k = Ws.shape[0]
    A_s = (Ws.reshape(k, _H, _C) * a_s[None]).sum(-1)
    A_d = (Wd.reshape(k, _H, _C) * a_d[None]).sum(-1)
    ab = jnp.tile(jnp.concatenate([A_s, A_d], axis=1), (1, 16))
    return jnp.concatenate([Ws, Wl, ab], axis=1)


def kernel(x, edge_index, W_rna, b_rna, W_prot, b_prot, Ws1, Wd1, as1, ad1,
           bg1, Wl1, bl1, Ws2, Wd2, as2, ad2, bg2, Wl2, bl2, Wagg, bagg,
           Wrd, brd, Wpd, bpd, Wrr, brr, Wpr, bpr):
    src = edge_index[0]
    dst = edge_index[1]
    r2 = lambda v: v.reshape(1, -1)
    f32 = jnp.float32

    z = jnp.zeros((_RNA, _EMB), f32)
    We = jnp.concatenate([
        jnp.concatenate([W_rna, z], axis=1),
        jnp.concatenate([z, W_prot], axis=1)], axis=0)
    be = r2(jnp.concatenate([b_rna, b_prot]))
    Wb1 = _proj_weights(Ws1, Wl1, as1, ad1, Wd1)
    Wb2 = _proj_weights(Ws2, Wl2, as2, ad2, Wd2)
    Wdcat = jnp.concatenate([Wrd, Wpd], axis=1)
    bdcat = r2(jnp.concatenate([brd, bpd]))
    zr = jnp.zeros((_EMB, _RNA), f32)
    zp = jnp.zeros((_EMB, _PROT), f32)
    Wrec = jnp.concatenate([
        jnp.concatenate([Wrr, zp], axis=1),
        jnp.concatenate([zr, Wpr], axis=1)], axis=0)
    brec = r2(jnp.concatenate([brr, bpr]))

    zn = jnp.zeros((3200, 32), f32)
    zd = jnp.zeros((6400,), f32)
    ab1, xsp1, hl1, mx1 = _tc1(x, We, be, Wb1, r2(bl1))
    gat1 = _sc_layer(src, dst, ab1, xsp1.reshape(4 * _N, 32),
                     _mt_from_mx(mx1), zn, zd)

    ab2, xsp2, hl2, mx2 = _tc2(gat1, hl1, r2(bg1), Wb2, r2(bl2))
    gat2 = _sc_layer(src, dst, ab2, xsp2.reshape(4 * _N, 32),
                     _mt_from_mx(mx2), zn, zd)

    rr, pr, emb = _tc3(gat2, hl2, r2(bg2), Wagg, r2(bagg), Wdcat, bdcat,
                       Wrec, brec)
    return (rr, pr, emb)
